# baseline (speedup 1.0000x reference)
import jax
import jax.numpy as jnp
from jax import lax
from jax.experimental import pallas as pl
from jax.experimental.pallas import tpu as pltpu

_LENS = (4096, 3584, 3072, 2560, 2048, 1536, 1024, 512)
_MAX = 4096
_NC = 5


def _coeffs():
    col = lax.broadcasted_iota(jnp.int32, (1, _NC), 1)
    xy = (col == 1) | (col == 2)
    mins = jnp.where(xy, -100.0, jnp.where(col == 3, -10.0, 0.0))
    inv = jnp.where(
        xy, 1.0 / 200.0,
        jnp.where(col == 3, 1.0 / 20.0, jnp.where(col == 4, 1.0 / 255.0, 1.0)))
    return mins.astype(jnp.float32), inv.astype(jnp.float32)


def _body(*refs):
    (*seqs, out_ref) = refs
    i = pl.program_id(0)
    mins, inv = _coeffs()
    for w in range(8):
        @pl.when(i == w)
        def _(w=w):
            x = seqs[w][...]
            n = (x - mins) * inv
            L = _LENS[w]
            out_ref[0, :L, :] = n
            if L < _MAX:
                out_ref[0, L:, :] = jnp.full((_MAX - L, _NC), -1.0,
                                             jnp.float32)


def kernel(seq0, seq1, seq2, seq3, seq4, seq5, seq6, seq7):
    seqs = (seq0, seq1, seq2, seq3, seq4, seq5, seq6, seq7)
    in_specs = [
        pl.BlockSpec((L, _NC), lambda i: (0, 0)) for L in _LENS
    ]
    out_spec = pl.BlockSpec((1, _MAX, _NC), lambda i: (i, 0, 0))
    return pl.pallas_call(
        _body,
        grid=(8,),
        in_specs=in_specs,
        out_specs=out_spec,
        out_shape=jax.ShapeDtypeStruct((8, _MAX, _NC), jnp.float32),
        compiler_params=pltpu.CompilerParams(
            dimension_semantics=("arbitrary",)),
    )(*seqs)
